# Initial kernel scaffold; baseline (speedup 1.0000x reference)
import jax
import jax.numpy as jnp
from jax.experimental import pallas as pl


def kernel(x, edge_index, edge_attr):
    raise NotImplementedError("write your pallas kernel here")


# speedup vs baseline: 165.0563x; 165.0563x over previous
import functools

import jax
import jax.numpy as jnp
from jax import lax
from jax.experimental import pallas as pl
from jax.experimental.pallas import tpu as pltpu
from jax.experimental.pallas import tpu_sc as plsc

NC = 2
NS = 16
L = 16
NW = NC * NS

ROW_W = 128
ROWS_PER_CHUNK = 16
CHUNK = ROWS_PER_CHUNK * ROW_W


@functools.cache
def _sc_edge_kernel(n_nodes: int, n_acc: int, rows16: int, chunks_per_tile: int):
    rows_per_tile = rows16 // NW
    zslice = n_acc // NS

    mesh = plsc.VectorSubcoreMesh(core_axis_name="c", subcore_axis_name="s")

    @functools.partial(
        pl.kernel,
        mesh=mesh,
        compiler_params=pltpu.CompilerParams(needs_layout_passes=False),
        out_type=jax.ShapeDtypeStruct((NC, n_acc), jnp.float32),
        scratch_types=[
            pltpu.VMEM((n_nodes,), jnp.float32),
            pltpu.VMEM((ROWS_PER_CHUNK, ROW_W), jnp.int32),
            pltpu.VMEM((ROWS_PER_CHUNK, ROW_W), jnp.int32),
            pltpu.VMEM((ROWS_PER_CHUNK, ROW_W), jnp.float32),
            pltpu.VMEM((ROWS_PER_CHUNK, ROW_W), jnp.float32),
            pltpu.VMEM_SHARED((n_acc,), jnp.float32),
        ],
    )
    def sc_kernel(nodes_hbm, row_hbm, col_hbm, ev_hbm, zeros_hbm, out_hbm,
                  nodes_v, rbuf, cbuf, ebuf, lbuf, acc_sh):
        c = lax.axis_index("c")
        s = lax.axis_index("s")
        wid = s * NC + c

        pltpu.sync_copy(nodes_hbm, nodes_v)
        pltpu.sync_copy(zeros_hbm.at[pl.ds(s * zslice, zslice)],
                        acc_sh.at[pl.ds(s * zslice, zslice)])
        plsc.subcore_barrier()

        base = wid * rows_per_tile

        @pl.loop(0, chunks_per_tile)
        def _chunk(ci):
            rb = base + ci * ROWS_PER_CHUNK
            pltpu.sync_copy(row_hbm.at[pl.ds(rb, ROWS_PER_CHUNK)], rbuf)
            pltpu.sync_copy(col_hbm.at[pl.ds(rb, ROWS_PER_CHUNK)], cbuf)
            pltpu.sync_copy(ev_hbm.at[pl.ds(rb, ROWS_PER_CHUNK)], ebuf)
            for j in range(ROWS_PER_CHUNK):
                for k in range(ROW_W // L):
                    sl = pl.ds(k * L, L)
                    ir = rbuf[j, sl]
                    ic = cbuf[j, sl]
                    e = ebuf[j, sl]
                    src = plsc.load_gather(nodes_v, [ir])
                    dst = plsc.load_gather(nodes_v, [ic])
                    m = (src * e) > 0
                    safe = jnp.where(m, e, jnp.float32(1.0))
                    lbuf[j, sl] = jnp.where(m, (dst - src) / safe,
                                            jnp.float32(0.0))
                pltpu.sync_copy(lbuf.at[j], acc_sh.at[cbuf.at[j]], add=True)

        plsc.subcore_barrier()

        @pl.when(s == 0)
        def _():
            pltpu.sync_copy(acc_sh, out_hbm.at[c])

    return sc_kernel


@functools.cache
def _tc_sum_kernel(n_acc: int):
    def body(p_ref, o_ref):
        o_ref[...] = p_ref[0] + p_ref[1]

    return pl.pallas_call(
        body,
        out_shape=jax.ShapeDtypeStruct((n_acc // 128, 128), jnp.float32),
    )


def kernel(x, edge_index, edge_attr):
    n = x.shape[0]
    e_cnt = edge_index.shape[1]

    nodes = x[:, 0]
    row = edge_index[0].astype(jnp.int32)
    col = edge_index[1].astype(jnp.int32)
    ev = edge_attr[:, 0]

    edges_per_tile = -(-e_cnt // (NW * CHUNK)) * CHUNK
    e_pad = NW * edges_per_tile
    pad = e_pad - e_cnt
    if pad:
        row = jnp.concatenate([row, jnp.zeros((pad,), jnp.int32)])
        col = jnp.concatenate([col, jnp.zeros((pad,), jnp.int32)])
        ev = jnp.concatenate([ev, jnp.zeros((pad,), jnp.float32)])

    n_acc = -(-n // 2048) * 2048
    rows16 = e_pad // ROW_W

    sc = _sc_edge_kernel(n, n_acc, rows16, edges_per_tile // CHUNK)
    partial = sc(nodes,
                 row.reshape(rows16, ROW_W),
                 col.reshape(rows16, ROW_W),
                 ev.reshape(rows16, ROW_W),
                 jnp.zeros((n_acc,), jnp.float32))

    summed = _tc_sum_kernel(n_acc)(partial.reshape(NC, n_acc // 128, 128))
    return summed.reshape(-1)[:n]


# speedup vs baseline: 222.9074x; 1.3505x over previous
import functools

import jax
import jax.numpy as jnp
from jax import lax
from jax.experimental import pallas as pl
from jax.experimental.pallas import tpu as pltpu
from jax.experimental.pallas import tpu_sc as plsc

NC = 2
NS = 16
L = 16
NW = NC * NS

ROW_W = 128
ROWS_PER_CHUNK = 16
CHUNK = ROWS_PER_CHUNK * ROW_W
NB = 3


@functools.cache
def _sc_edge_kernel(n_nodes: int, n_acc: int, rows16: int, chunks_per_tile: int):
    rows_per_tile = rows16 // NW
    zslice = n_acc // NS

    mesh = plsc.VectorSubcoreMesh(core_axis_name="c", subcore_axis_name="s")

    idx_buf = pltpu.VMEM((ROWS_PER_CHUNK, ROW_W), jnp.int32)
    val_buf = pltpu.VMEM((ROWS_PER_CHUNK, ROW_W), jnp.float32)

    @functools.partial(
        pl.kernel,
        mesh=mesh,
        compiler_params=pltpu.CompilerParams(needs_layout_passes=False),
        out_type=jax.ShapeDtypeStruct((NC, n_acc), jnp.float32),
        scratch_types=[
            pltpu.VMEM((n_nodes,), jnp.float32),
            [idx_buf] * NB,
            [idx_buf] * NB,
            [val_buf] * NB,
            [val_buf] * NB,
            pltpu.VMEM_SHARED((n_acc,), jnp.float32),
            [pltpu.SemaphoreType.DMA] * NB,
            [pltpu.SemaphoreType.DMA] * NB,
        ],
    )
    def sc_kernel(nodes_hbm, row_hbm, col_hbm, ev_hbm, zeros_hbm, out_hbm,
                  nodes_v, rbufs, cbufs, ebufs, lbufs, acc_sh,
                  in_sems, sc_sems):
        c = lax.axis_index("c")
        s = lax.axis_index("s")
        wid = s * NC + c
        base_row = wid * rows_per_tile

        def in_descs(ci, slot):
            rb = base_row + ci * ROWS_PER_CHUNK
            sl = pl.ds(rb, ROWS_PER_CHUNK)
            sem = in_sems[slot]
            return (
                pltpu.make_async_copy(row_hbm.at[sl], rbufs[slot], sem),
                pltpu.make_async_copy(col_hbm.at[sl], cbufs[slot], sem),
                pltpu.make_async_copy(ev_hbm.at[sl], ebufs[slot], sem),
            )

        def sc_desc(slot, j):
            return pltpu.make_async_copy(
                lbufs[slot].at[j], acc_sh.at[cbufs[slot].at[j]],
                sc_sems[slot])

        pltpu.sync_copy(nodes_hbm, nodes_v)
        pltpu.sync_copy(zeros_hbm.at[pl.ds(s * zslice, zslice)],
                        acc_sh.at[pl.ds(s * zslice, zslice)])
        plsc.subcore_barrier()

        for d in in_descs(0, 0):
            d.start()

        @pl.loop(0, chunks_per_tile, step=NB)
        def _group(bi):
            for p in range(NB):
                ci = bi + p
                dslot = (p + 1) % NB
                if p == NB - 1:
                    for j in range(ROWS_PER_CHUNK):
                        sc_desc(dslot, j).wait()
                else:

                    @pl.when(bi >= NB)
                    def _():
                        for j in range(ROWS_PER_CHUNK):
                            sc_desc(dslot, j).wait()

                if p == NB - 1:

                    @pl.when(bi + NB < chunks_per_tile)
                    def _():
                        for d in in_descs(ci + 1, dslot):
                            d.start()
                else:
                    for d in in_descs(ci + 1, dslot):
                        d.start()

                for d in in_descs(ci, p):
                    d.wait()
                for j in range(ROWS_PER_CHUNK):
                    for k in range(ROW_W // L):
                        sl = pl.ds(k * L, L)
                        ir = rbufs[p][j, sl]
                        ic = cbufs[p][j, sl]
                        e = ebufs[p][j, sl]
                        src = plsc.load_gather(nodes_v, [ir])
                        dst = plsc.load_gather(nodes_v, [ic])
                        m = (src * e) > 0
                        safe = jnp.where(m, e, jnp.float32(1.0))
                        lbufs[p][j, sl] = jnp.where(m, (dst - src) / safe,
                                                    jnp.float32(0.0))
                    sc_desc(p, j).start(add=True)

        for slot in (1, 2):
            for j in range(ROWS_PER_CHUNK):
                sc_desc(slot, j).wait()

        plsc.subcore_barrier()

        @pl.when(s == 0)
        def _():
            pltpu.sync_copy(acc_sh, out_hbm.at[c])

    return sc_kernel


@functools.cache
def _tc_sum_kernel(n_acc: int):
    def body(p_ref, o_ref):
        o_ref[...] = p_ref[0] + p_ref[1]

    return pl.pallas_call(
        body,
        out_shape=jax.ShapeDtypeStruct((n_acc // 128, 128), jnp.float32),
    )


def kernel(x, edge_index, edge_attr):
    n = x.shape[0]
    e_cnt = edge_index.shape[1]

    nodes = x[:, 0]
    row = edge_index[0].astype(jnp.int32)
    col = edge_index[1].astype(jnp.int32)
    ev = edge_attr[:, 0]

    grain = NW * CHUNK * NB
    e_pad = -(-e_cnt // grain) * grain
    pad = e_pad - e_cnt
    if pad:
        row = jnp.concatenate([row, jnp.zeros((pad,), jnp.int32)])
        col = jnp.concatenate([col, jnp.zeros((pad,), jnp.int32)])
        ev = jnp.concatenate([ev, jnp.zeros((pad,), jnp.float32)])

    n_acc = -(-n // 2048) * 2048
    rows16 = e_pad // ROW_W

    sc = _sc_edge_kernel(n, n_acc, rows16, e_pad // (NW * CHUNK))
    partial = sc(nodes,
                 row.reshape(rows16, ROW_W),
                 col.reshape(rows16, ROW_W),
                 ev.reshape(rows16, ROW_W),
                 jnp.zeros((n_acc,), jnp.float32))

    summed = _tc_sum_kernel(n_acc)(partial.reshape(NC, n_acc // 128, 128))
    return summed.reshape(-1)[:n]
